# baseline (speedup 1.0000x reference)
import functools
import math

import jax
import jax.numpy as jnp
import numpy as np
from jax.experimental import pallas as pl
from jax.experimental.pallas import tpu as pltpu

E = 16
W_IMPORTANCE = 0.01
W_LOAD = 0.01
LAMBDA_Z = 0.001
W_PENALTY = 0.01
INV_TEMP = 1.0 / 1.66


def _gelu_tanh(h):
    return 0.5 * h * (1.0 + jnp.tanh(np.sqrt(2.0 / np.pi) * (h + 0.044715 * h ** 3)))


def _router_kernel(x_ref, wg_ref, bgb_ref, g_ref, sel_ref, stats_ref, aux_ref,
                   *, nb, s_total):
    i = pl.program_id(0)
    x = x_ref[...]
    wg = wg_ref[...]
    guess = jax.lax.dot_general(x, wg, (((1,), (1,)), ((), ())),
                                preferred_element_type=jnp.float32)
    guess = guess + bgb_ref[...]

    m = jnp.max(guess, axis=1, keepdims=True)
    ex = jnp.exp(guess - m)
    sex = jnp.sum(ex, axis=1, keepdims=True)
    lse = m + jnp.log(sex)
    lse2_sum = jnp.sum(lse * lse)

    logits = guess * INV_TEMP
    lm = jnp.max(logits, axis=1, keepdims=True)
    lex = jnp.exp(logits - lm)
    prob = lex / jnp.sum(lex, axis=1, keepdims=True)

    sel = jnp.argmax(logits + g_ref[...], axis=1).astype(jnp.int32)
    sel_ref[0, 0, :] = sel

    onehot = (sel[:, None] ==
              jax.lax.broadcasted_iota(jnp.int32, (1, E), 1)).astype(jnp.float32)

    psum = jnp.sum(prob, axis=0, keepdims=True)
    counts = jnp.sum(onehot, axis=0, keepdims=True)
    weights = jnp.sum(prob * onehot, axis=1, keepdims=True)
    imp = jnp.sum(onehot * weights, axis=0, keepdims=True)
    pq_sum = jnp.sum(prob * (1.0 - prob))

    lane = jax.lax.broadcasted_iota(jnp.int32, (1, E), 1)
    misc = (jnp.where(lane == 0, lse2_sum, 0.0)
            + jnp.where(lane == 1, pq_sum, 0.0))
    partial = jnp.concatenate(
        [misc, psum, counts, imp,
         jnp.zeros((4, E), dtype=jnp.float32)], axis=0)

    @pl.when(i == 0)
    def _init():
        stats_ref[...] = jnp.zeros_like(stats_ref)
        aux_ref[...] = jnp.zeros_like(aux_ref)

    stats_ref[...] += partial

    @pl.when(i == nb - 1)
    def _finish():
        s = stats_ref[...]
        lane2 = jax.lax.broadcasted_iota(jnp.int32, (1, E), 1)
        row_misc = s[0:1, :]
        lse2_tot = jnp.sum(jnp.where(lane2 == 0, row_misc, 0.0))
        pq_tot = jnp.sum(jnp.where(lane2 == 1, row_misc, 0.0))
        p_i = s[1:2, :] / s_total
        f_i = s[2:3, :] / s_total
        impv = s[3:4, :]
        z_loss = LAMBDA_Z * lse2_tot / s_total
        load_loss = W_LOAD * E * jnp.sum(f_i * p_i)
        imp_mean = jnp.mean(impv)
        imp_var = jnp.mean((impv - imp_mean) ** 2)
        imp_loss = W_IMPORTANCE * imp_var / (imp_mean * imp_mean)
        pen_a = pq_tot / (s_total * E)
        pen_b = 1.0 / E - jnp.mean(p_i * (1.0 - p_i))
        penalty = W_PENALTY * (pen_a + pen_b)
        aux = z_loss + load_loss + imp_loss + penalty
        aux_ref[...] = aux * jnp.ones((1, 1), jnp.float32)


def _mlp_kernel(x_ref, wfc_ref, bfc_ref, wproj_ref, bproj_ref, sel_ref, y_ref,
                *, nf):
    j = pl.program_id(1)
    sel = sel_ref[0, 0, :]
    onehot = (sel[:, None] ==
              jax.lax.broadcasted_iota(jnp.int32, (1, E), 1)).astype(jnp.float32)

    x = x_ref[...]
    h = jax.lax.dot_general(x, wfc_ref[...], (((1,), (1,)), ((), ())),
                            preferred_element_type=jnp.float32)
    h = h + jax.lax.dot_general(onehot, bfc_ref[...], (((1,), (0,)), ((), ())),
                                preferred_element_type=jnp.float32)
    h = _gelu_tanh(h)
    yp = jax.lax.dot_general(h, wproj_ref[...], (((1,), (1,)), ((), ())),
                             preferred_element_type=jnp.float32)

    @pl.when(j == 0)
    def _first():
        y_ref[...] = yp + jax.lax.dot_general(
            onehot, bproj_ref[...], (((1,), (0,)), ((), ())),
            preferred_element_type=jnp.float32)

    @pl.when(j != 0)
    def _rest():
        y_ref[...] += yp


def kernel(x, Wg, bg, bias, Wfc, Bfc, Wproj, Bproj):
    B, T, C = x.shape
    F = Wfc.shape[0]
    S = B * T
    xs = x.reshape(S, C)

    g = jax.random.gumbel(jax.random.key(42), (B, T, E), jnp.float32)
    g = g.reshape(S, E)
    bgb = (bg + bias).reshape(1, E)

    TB = 2048
    nb = S // TB
    sel3d, stats, aux = pl.pallas_call(
        functools.partial(_router_kernel, nb=nb, s_total=float(S)),
        grid=(nb,),
        in_specs=[
            pl.BlockSpec((TB, C), lambda i: (i, 0)),
            pl.BlockSpec((E, C), lambda i: (0, 0)),
            pl.BlockSpec((1, E), lambda i: (0, 0)),
            pl.BlockSpec((TB, E), lambda i: (i, 0)),
        ],
        out_specs=[
            pl.BlockSpec((1, 1, TB), lambda i: (i, 0, 0)),
            pl.BlockSpec((8, E), lambda i: (0, 0)),
            pl.BlockSpec((1, 1), lambda i: (0, 0)),
        ],
        out_shape=[
            jax.ShapeDtypeStruct((nb, 1, TB), jnp.int32),
            jax.ShapeDtypeStruct((8, E), jnp.float32),
            jax.ShapeDtypeStruct((1, 1), jnp.float32),
        ],
        compiler_params=pltpu.CompilerParams(
            dimension_semantics=("arbitrary",)),
    )(xs, Wg, bgb, g)

    TBM = 512
    FB = 1024
    nt = S // TBM
    nf = F // FB
    sel_mlp = sel3d.reshape(S).reshape(nt, 1, TBM)

    y = pl.pallas_call(
        functools.partial(_mlp_kernel, nf=nf),
        grid=(nt, nf),
        in_specs=[
            pl.BlockSpec((TBM, C), lambda i, j: (i, 0)),
            pl.BlockSpec((FB, C), lambda i, j: (j, 0)),
            pl.BlockSpec((E, FB), lambda i, j: (0, j)),
            pl.BlockSpec((C, FB), lambda i, j: (0, j)),
            pl.BlockSpec((E, C), lambda i, j: (0, 0)),
            pl.BlockSpec((1, 1, TBM), lambda i, j: (i, 0, 0)),
        ],
        out_specs=pl.BlockSpec((TBM, C), lambda i, j: (i, 0)),
        out_shape=jax.ShapeDtypeStruct((S, C), jnp.float32),
        compiler_params=pltpu.CompilerParams(
            dimension_semantics=("parallel", "arbitrary")),
    )(xs, Wfc, Bfc, Wproj, Bproj, sel_mlp)

    return (y.reshape(B, T, C), aux[0, 0])


# speedup vs baseline: 1.0635x; 1.0635x over previous
import functools
import math

import jax
import jax.numpy as jnp
import numpy as np
from jax.experimental import pallas as pl
from jax.experimental.pallas import tpu as pltpu

E = 16
W_IMPORTANCE = 0.01
W_LOAD = 0.01
LAMBDA_Z = 0.001
W_PENALTY = 0.01
INV_TEMP = 1.0 / 1.66


def _gelu_tanh(h):
    return 0.5 * h * (1.0 + jnp.tanh(np.sqrt(2.0 / np.pi) * (h + 0.044715 * h ** 3)))


def _router_kernel(x_ref, wg_ref, bgb_ref, g_ref, sel_ref, stats_ref, aux_ref,
                   *, nb, s_total):
    i = pl.program_id(0)
    x = x_ref[...]
    wg = wg_ref[...]
    guess = jax.lax.dot_general(x, wg, (((1,), (1,)), ((), ())),
                                preferred_element_type=jnp.float32)
    guess = guess + bgb_ref[...]

    m = jnp.max(guess, axis=1, keepdims=True)
    ex = jnp.exp(guess - m)
    sex = jnp.sum(ex, axis=1, keepdims=True)
    lse = m + jnp.log(sex)
    lse2_sum = jnp.sum(lse * lse)

    logits = guess * INV_TEMP
    lm = jnp.max(logits, axis=1, keepdims=True)
    lex = jnp.exp(logits - lm)
    prob = lex / jnp.sum(lex, axis=1, keepdims=True)

    sel = jnp.argmax(logits + g_ref[...], axis=1).astype(jnp.int32)
    sel_ref[0, 0, :] = sel

    onehot = (sel[:, None] ==
              jax.lax.broadcasted_iota(jnp.int32, (1, E), 1)).astype(jnp.float32)

    psum = jnp.sum(prob, axis=0, keepdims=True)
    counts = jnp.sum(onehot, axis=0, keepdims=True)
    weights = jnp.sum(prob * onehot, axis=1, keepdims=True)
    imp = jnp.sum(onehot * weights, axis=0, keepdims=True)
    pq_sum = jnp.sum(prob * (1.0 - prob))

    lane = jax.lax.broadcasted_iota(jnp.int32, (1, E), 1)
    misc = (jnp.where(lane == 0, lse2_sum, 0.0)
            + jnp.where(lane == 1, pq_sum, 0.0))
    partial = jnp.concatenate(
        [misc, psum, counts, imp,
         jnp.zeros((4, E), dtype=jnp.float32)], axis=0)

    @pl.when(i == 0)
    def _init():
        stats_ref[...] = jnp.zeros_like(stats_ref)
        aux_ref[...] = jnp.zeros_like(aux_ref)

    stats_ref[...] += partial

    @pl.when(i == nb - 1)
    def _finish():
        s = stats_ref[...]
        lane2 = jax.lax.broadcasted_iota(jnp.int32, (1, E), 1)
        row_misc = s[0:1, :]
        lse2_tot = jnp.sum(jnp.where(lane2 == 0, row_misc, 0.0))
        pq_tot = jnp.sum(jnp.where(lane2 == 1, row_misc, 0.0))
        p_i = s[1:2, :] / s_total
        f_i = s[2:3, :] / s_total
        impv = s[3:4, :]
        z_loss = LAMBDA_Z * lse2_tot / s_total
        load_loss = W_LOAD * E * jnp.sum(f_i * p_i)
        imp_mean = jnp.mean(impv)
        imp_var = jnp.mean((impv - imp_mean) ** 2)
        imp_loss = W_IMPORTANCE * imp_var / (imp_mean * imp_mean)
        pen_a = pq_tot / (s_total * E)
        pen_b = 1.0 / E - jnp.mean(p_i * (1.0 - p_i))
        penalty = W_PENALTY * (pen_a + pen_b)
        aux = z_loss + load_loss + imp_loss + penalty
        aux_ref[...] = aux * jnp.ones((1, 1), jnp.float32)


def _mlp_kernel(x_ref, wfc_ref, bfc_ref, wproj_ref, bproj_ref, sel_ref, y_ref,
                *, nf):
    j = pl.program_id(1)
    sel = sel_ref[0, 0, :]
    onehot = (sel[:, None] ==
              jax.lax.broadcasted_iota(jnp.int32, (1, E), 1)).astype(jnp.float32)

    x = x_ref[...]
    h = jax.lax.dot_general(x, wfc_ref[...], (((1,), (1,)), ((), ())),
                            preferred_element_type=jnp.float32)
    h = h + jax.lax.dot_general(onehot, bfc_ref[...], (((1,), (0,)), ((), ())),
                                preferred_element_type=jnp.float32)
    h = _gelu_tanh(h)

    @pl.when(j == 0)
    def _first():
        y_ref[...] = jax.lax.dot_general(
            onehot, bproj_ref[...], (((1,), (0,)), ((), ())),
            preferred_element_type=jnp.float32)

    y_ref[...] += jax.lax.dot_general(h, wproj_ref[...], (((1,), (1,)), ((), ())),
                                      preferred_element_type=jnp.float32)


def kernel(x, Wg, bg, bias, Wfc, Bfc, Wproj, Bproj):
    B, T, C = x.shape
    F = Wfc.shape[0]
    S = B * T
    xs = x.reshape(S, C)

    g = jax.random.gumbel(jax.random.key(42), (B, T, E), jnp.float32)
    g = g.reshape(S, E)
    bgb = (bg + bias).reshape(1, E)

    TB = 2048
    nb = S // TB
    sel3d, stats, aux = pl.pallas_call(
        functools.partial(_router_kernel, nb=nb, s_total=float(S)),
        grid=(nb,),
        in_specs=[
            pl.BlockSpec((TB, C), lambda i: (i, 0)),
            pl.BlockSpec((E, C), lambda i: (0, 0)),
            pl.BlockSpec((1, E), lambda i: (0, 0)),
            pl.BlockSpec((TB, E), lambda i: (i, 0)),
        ],
        out_specs=[
            pl.BlockSpec((1, 1, TB), lambda i: (i, 0, 0)),
            pl.BlockSpec((8, E), lambda i: (0, 0)),
            pl.BlockSpec((1, 1), lambda i: (0, 0)),
        ],
        out_shape=[
            jax.ShapeDtypeStruct((nb, 1, TB), jnp.int32),
            jax.ShapeDtypeStruct((8, E), jnp.float32),
            jax.ShapeDtypeStruct((1, 1), jnp.float32),
        ],
        compiler_params=pltpu.CompilerParams(
            dimension_semantics=("arbitrary",)),
    )(xs, Wg, bgb, g)

    TBM = 1024
    FB = 512
    nt = S // TBM
    nf = F // FB
    sel_mlp = sel3d.reshape(S).reshape(nt, 1, TBM)

    y = pl.pallas_call(
        functools.partial(_mlp_kernel, nf=nf),
        grid=(nt, nf),
        in_specs=[
            pl.BlockSpec((TBM, C), lambda i, j: (i, 0)),
            pl.BlockSpec((FB, C), lambda i, j: (j, 0)),
            pl.BlockSpec((E, FB), lambda i, j: (0, j)),
            pl.BlockSpec((C, FB), lambda i, j: (0, j)),
            pl.BlockSpec((E, C), lambda i, j: (0, 0)),
            pl.BlockSpec((1, 1, TBM), lambda i, j: (i, 0, 0)),
        ],
        out_specs=pl.BlockSpec((TBM, C), lambda i, j: (i, 0)),
        out_shape=jax.ShapeDtypeStruct((S, C), jnp.float32),
        compiler_params=pltpu.CompilerParams(
            dimension_semantics=("parallel", "arbitrary")),
    )(xs, Wfc, Bfc, Wproj, Bproj, sel_mlp)

    return (y.reshape(B, T, C), aux[0, 0])
